# baseline (speedup 1.0000x reference)
import functools

import jax
import jax.numpy as jnp
from jax import lax
from jax.experimental import pallas as pl
from jax.experimental.pallas import tpu as pltpu
from jax.experimental.pallas import tpu_sc as plsc


@functools.lru_cache(maxsize=None)
def _make_gather(V, D, B):
    info = plsc.get_sparse_core_info()
    NC, NS, L = info.num_cores, info.num_subcores, info.num_lanes
    NW = NC * NS
    assert B % NW == 0
    b_per_w = B // NW
    CHUNK = 128
    assert b_per_w % CHUNK == 0
    n_chunks = b_per_w // CHUNK
    DP = 2 * 128
    NBUF = 3

    mesh = plsc.VectorSubcoreMesh(core_axis_name="c", subcore_axis_name="s")

    @functools.partial(
        pl.kernel,
        mesh=mesh,
        compiler_params=pltpu.CompilerParams(use_tc_tiling_on_sc=False),
        out_type=jax.ShapeDtypeStruct((B, D), jnp.float32),
        scratch_types=[
            pltpu.VMEM((b_per_w,), jnp.int32),
            pltpu.VMEM((CHUNK, D), jnp.float32),
            pltpu.VMEM((CHUNK, D), jnp.float32),
            pltpu.VMEM((CHUNK, D), jnp.float32),
            pltpu.SemaphoreType.DMA,
            pltpu.SemaphoreType.DMA,
        ],
    )
    def k(table_hbm, idx_hbm, out_hbm,
          idx_v, buf0_v, buf1_v, buf2_v, sem_g, sem_w):
        wid = lax.axis_index("s") * NC + lax.axis_index("c")
        base = wid * b_per_w
        pltpu.sync_copy(idx_hbm.at[pl.ds(base, b_per_w)], idx_v)

        row_view = table_hbm
        bufs = [buf0_v, buf1_v, buf2_v]

        gathers = [None] * n_chunks
        writes = [None] * n_chunks

        def fire(j):
            gathers[j] = pltpu.async_copy(
                row_view.at[idx_v.at[pl.ds(j * CHUNK, CHUNK)]],
                bufs[j % NBUF], sem_g)

        for j in range(min(NBUF, n_chunks)):
            fire(j)
        for j in range(n_chunks):
            gathers[j].wait()
            writes[j] = pltpu.async_copy(
                bufs[j % NBUF],
                out_hbm.at[pl.ds(base + j * CHUNK, CHUNK)], sem_w)
            nxt = j + NBUF
            if nxt < n_chunks:
                writes[j].wait()
                fire(nxt)
        for j in range(max(0, n_chunks - NBUF), n_chunks):
            writes[j].wait()

    return k


def kernel(batch, emb_weight):
    V, D = emb_weight.shape
    (B,) = batch.shape
    return _make_gather(V, D, B)(emb_weight, batch.astype(jnp.int32))


# speedup vs baseline: 4.8916x; 4.8916x over previous
import functools

import jax
import jax.numpy as jnp
from jax import lax
from jax.experimental import pallas as pl
from jax.experimental.pallas import tpu as pltpu
from jax.experimental.pallas import tpu_sc as plsc


@functools.lru_cache(maxsize=None)
def _make_gather(V, D, B):
    info = plsc.get_sparse_core_info()
    NC, NS, L = info.num_cores, info.num_subcores, info.num_lanes
    NW = NC * NS
    assert B % NW == 0
    b_per_w = B // NW
    CHUNK = 128
    assert b_per_w % CHUNK == 0
    n_chunks = b_per_w // CHUNK
    DM = D - 1
    NBUF = 3

    mesh = plsc.VectorSubcoreMesh(core_axis_name="c", subcore_axis_name="s")

    @functools.partial(
        pl.kernel,
        mesh=mesh,
        out_type=(
            jax.ShapeDtypeStruct((B, DM), jnp.float32),
            jax.ShapeDtypeStruct((B,), jnp.float32),
        ),
        scratch_types=[
            pltpu.VMEM((b_per_w,), jnp.int32),
            pltpu.VMEM((CHUNK, DM), jnp.float32),
            pltpu.VMEM((CHUNK, DM), jnp.float32),
            pltpu.VMEM((CHUNK, DM), jnp.float32),
            pltpu.VMEM((b_per_w,), jnp.float32),
            pltpu.SemaphoreType.DMA,
            pltpu.SemaphoreType.DMA,
            pltpu.SemaphoreType.DMA,
        ],
    )
    def k(main_hbm, last_hbm, idx_hbm, out_main_hbm, out_last_hbm,
          idx_v, buf0_v, buf1_v, buf2_v, last_v, sem_g, sem_w, sem_l):
        wid = lax.axis_index("s") * NC + lax.axis_index("c")
        base = wid * b_per_w
        pltpu.sync_copy(idx_hbm.at[pl.ds(base, b_per_w)], idx_v)

        bufs = [buf0_v, buf1_v, buf2_v]
        gathers = [None] * n_chunks
        writes = [None] * n_chunks

        last_copies = [
            pltpu.async_copy(
                last_hbm.at[idx_v.at[pl.ds(j * CHUNK, CHUNK)]],
                last_v.at[pl.ds(j * CHUNK, CHUNK)], sem_l)
            for j in range(n_chunks)
        ]

        def fire(j):
            gathers[j] = pltpu.async_copy(
                main_hbm.at[idx_v.at[pl.ds(j * CHUNK, CHUNK)]],
                bufs[j % NBUF], sem_g)

        for j in range(min(NBUF, n_chunks)):
            fire(j)
        for j in range(n_chunks):
            gathers[j].wait()
            writes[j] = pltpu.async_copy(
                bufs[j % NBUF],
                out_main_hbm.at[pl.ds(base + j * CHUNK, CHUNK)], sem_w)
            nxt = j + NBUF
            if nxt < n_chunks:
                writes[j].wait()
                fire(nxt)
        for j in range(max(0, n_chunks - NBUF), n_chunks):
            writes[j].wait()
        for c in last_copies:
            c.wait()
        pltpu.sync_copy(last_v, out_last_hbm.at[pl.ds(base, b_per_w)])

    return k


def kernel(batch, emb_weight):
    V, D = emb_weight.shape
    (B,) = batch.shape
    main, last = _make_gather(V, D, B)(
        emb_weight[:, : D - 1],
        emb_weight[:, D - 1],
        batch.astype(jnp.int32),
    )
    return jnp.concatenate([main, last[:, None]], axis=1)


# speedup vs baseline: 5.7643x; 1.1784x over previous
import functools

import jax
import jax.numpy as jnp
from jax import lax
from jax.experimental import pallas as pl
from jax.experimental.pallas import tpu as pltpu
from jax.experimental.pallas import tpu_sc as plsc


@functools.lru_cache(maxsize=None)
def _make_gather(V, D, B):
    info = plsc.get_sparse_core_info()
    NC, NS, L = info.num_cores, info.num_subcores, info.num_lanes
    NW = NC * NS
    assert B % NW == 0
    b_per_w = B // NW
    CHUNK = 64
    assert b_per_w % CHUNK == 0
    n_chunks = b_per_w // CHUNK
    DP = 2 * 128
    NBUF = 6

    mesh = plsc.VectorSubcoreMesh(core_axis_name="c", subcore_axis_name="s")

    @functools.partial(
        pl.kernel,
        mesh=mesh,
        out_type=jax.ShapeDtypeStruct((B, D), jnp.float32),
        scratch_types=[
            pltpu.VMEM((b_per_w,), jnp.int32),
            pltpu.VMEM((CHUNK, D), jnp.float32),
            pltpu.VMEM((CHUNK, D), jnp.float32),
            pltpu.VMEM((CHUNK, D), jnp.float32),
            pltpu.VMEM((CHUNK, D), jnp.float32),
            pltpu.VMEM((CHUNK, D), jnp.float32),
            pltpu.VMEM((CHUNK, D), jnp.float32),
            pltpu.SemaphoreType.DMA,
            pltpu.SemaphoreType.DMA,
        ],
    )
    def k(table_hbm, idx_hbm, out_hbm,
          idx_v, buf0_v, buf1_v, buf2_v, buf3_v, buf4_v, buf5_v,
          sem_g, sem_w):
        wid = lax.axis_index("s") * NC + lax.axis_index("c")
        base = wid * b_per_w
        pltpu.sync_copy(idx_hbm.at[pl.ds(base, b_per_w)], idx_v)

        start = pl.multiple_of(wid * 0, DP)
        row_view = table_hbm.at[:, pl.ds(start, DP)]
        bufs = [buf0_v, buf1_v, buf2_v, buf3_v, buf4_v, buf5_v]

        gathers = [None] * n_chunks
        writes = [None] * n_chunks

        def fire(j):
            gathers[j] = pltpu.async_copy(
                row_view.at[idx_v.at[pl.ds(j * CHUNK, CHUNK)]],
                bufs[j % NBUF].at[:, pl.ds(start, DP)], sem_g)

        for j in range(min(NBUF, n_chunks)):
            fire(j)
        for j in range(n_chunks):
            gathers[j].wait()
            writes[j] = pltpu.async_copy(
                bufs[j % NBUF],
                out_hbm.at[pl.ds(base + j * CHUNK, CHUNK)], sem_w)
            nxt = j + NBUF
            if nxt < n_chunks:
                writes[j].wait()
                fire(nxt)
        for j in range(max(0, n_chunks - NBUF), n_chunks):
            writes[j].wait()

    return k


def kernel(batch, emb_weight):
    V, D = emb_weight.shape
    (B,) = batch.shape
    return _make_gather(V, D, B)(emb_weight, batch.astype(jnp.int32))
